# Initial kernel scaffold; baseline (speedup 1.0000x reference)
import jax
import jax.numpy as jnp
from jax.experimental import pallas as pl


def kernel(feature_ids, feature_values, type_embed_table, value_proj_W, value_proj_b):
    raise NotImplementedError("write your pallas kernel here")


# speedup vs baseline: 4.3914x; 4.3914x over previous
import functools

import jax
import jax.numpy as jnp
from jax import lax
from jax.experimental import pallas as pl
from jax.experimental.pallas import tpu as pltpu
from jax.experimental.pallas import tpu_sc as plsc

NC = 2
NS = 16
NW = NC * NS
L = 16

N_FEAT = 6
D = 16
CHUNK = 2048


@functools.lru_cache(maxsize=None)
def _build(n_tokens: int):
    tok_per_w = n_tokens // NW
    n_chunks = tok_per_w // CHUNK
    assert tok_per_w * NW == n_tokens and n_chunks * CHUNK == tok_per_w

    mesh = plsc.VectorSubcoreMesh(
        core_axis_name="c", subcore_axis_name="s", num_cores=NC, num_subcores=NS
    )

    @functools.partial(
        pl.kernel,
        out_type=jax.ShapeDtypeStruct((n_tokens * D,), jnp.float32),
        mesh=mesh,
        scratch_types=[
            pltpu.VMEM((N_FEAT * D,), jnp.float32),
            pltpu.VMEM((N_FEAT * D,), jnp.float32),
            pltpu.VMEM((D * D,), jnp.float32),
            pltpu.VMEM((D,), jnp.float32),
            pltpu.VMEM((CHUNK,), jnp.int32),
            pltpu.VMEM((CHUNK,), jnp.float32),
            pltpu.VMEM((CHUNK * D,), jnp.float32),
        ],
        compiler_params=pltpu.CompilerParams(needs_layout_passes=False),
    )
    def run(ids_hbm, vals_hbm, tab_hbm, w_hbm, b_hbm, out_hbm,
            tab_v, tab2_v, w_v, b_v, ids_v, vals_v, out_v):
        wid = lax.axis_index("s") * NC + lax.axis_index("c")

        pltpu.sync_copy(tab_hbm, tab_v)
        pltpu.sync_copy(w_hbm, w_v)
        pltpu.sync_copy(b_hbm, b_v)

        bvec = b_v[...]
        for r in range(N_FEAT):
            tab2_v[pl.ds(r * D, D)] = tab_v[pl.ds(r * D, D)] + bvec
        wsplats = [w_v[pl.ds(d * D, D)] for d in range(D)]
        lane16 = lax.iota(jnp.int32, L) * D

        base_w = wid * tok_per_w

        def chunk_body(cidx, carry):
            tok0 = base_w + cidx * CHUNK
            pltpu.sync_copy(ids_hbm.at[pl.ds(tok0, CHUNK)], ids_v)
            pltpu.sync_copy(vals_hbm.at[pl.ds(tok0, CHUNK)], vals_v)

            def group_body(g, gcarry):
                ids_vec = ids_v[pl.ds(g * L, L)]
                vals_vec = vals_v[pl.ds(g * L, L)]
                gbase = ids_vec * D
                obase = lane16 + g * (L * D)
                for d in range(D):
                    col = plsc.load_gather(tab2_v, [gbase + d])
                    col = col + vals_vec * wsplats[d]
                    plsc.store_scatter(out_v, [obase + d], col)
                return gcarry

            lax.fori_loop(0, CHUNK // L, group_body, 0)
            pltpu.sync_copy(out_v, out_hbm.at[pl.ds(tok0 * D, CHUNK * D)])
            return carry

        lax.fori_loop(0, n_chunks, chunk_body, 0)

    return run


def kernel(feature_ids, feature_values, type_embed_table, value_proj_W, value_proj_b):
    batch, seq = feature_ids.shape
    n_tokens = batch * seq
    ids = feature_ids.reshape(n_tokens).astype(jnp.int32)
    vals = feature_values.reshape(n_tokens).astype(jnp.float32)
    tab = type_embed_table.reshape(N_FEAT * D).astype(jnp.float32)
    w = value_proj_W.reshape(D).astype(jnp.float32)
    wmat = jnp.broadcast_to(w[:, None], (D, D)).reshape(D * D)
    b = value_proj_b.reshape(D).astype(jnp.float32)
    out_flat = _build(n_tokens)(ids, vals, tab, wmat, b)
    return out_flat.reshape(batch, seq, D)


# speedup vs baseline: 11.4956x; 2.6178x over previous
import functools

import jax
import jax.numpy as jnp
from jax import lax
from jax.experimental import pallas as pl
from jax.experimental.pallas import tpu as pltpu
from jax.experimental.pallas import tpu_sc as plsc

NC = 2
NS = 16
NW = NC * NS
L = 16

N_FEAT = 6
D = 16
B_SZ = 16384
S_SZ = 200

BW = 4
CB = B_SZ // (BW * 1024)
CHUNK_B = BW * 1024
PLANE = D * B_SZ
HALF = CHUNK_B * 8

S_BIG = S_SZ - 6 * NW


@functools.lru_cache(maxsize=None)
def _build():
    mesh = plsc.VectorSubcoreMesh(
        core_axis_name="c", subcore_axis_name="s", num_cores=NC, num_subcores=NS
    )

    @functools.partial(
        pl.kernel,
        out_type=jax.ShapeDtypeStruct((S_SZ * PLANE,), jnp.float32),
        mesh=mesh,
        scratch_types=[
            pltpu.VMEM((N_FEAT * D,), jnp.float32),
            pltpu.VMEM((D * N_FEAT,), jnp.float32),
            pltpu.VMEM((D * D,), jnp.float32),
            pltpu.VMEM((D,), jnp.float32),
            pltpu.VMEM((CHUNK_B,), jnp.int32),
            pltpu.VMEM((CHUNK_B,), jnp.float32),
            pltpu.VMEM((2 * HALF,), jnp.float32),
        ],
        compiler_params=pltpu.CompilerParams(needs_layout_passes=False),
    )
    def run(ids_hbm, vals_hbm, tab_hbm, w_hbm, b_hbm, out_hbm,
            tab_v, tabt_v, w_v, b_v, ids_v, vals_v, out_v):
        wid = lax.axis_index("s") * NC + lax.axis_index("c")

        pltpu.sync_copy(tab_hbm, tab_v)
        pltpu.sync_copy(w_hbm, w_v)
        pltpu.sync_copy(b_hbm, b_v)

        bvec = b_v[...]
        lane = lax.iota(jnp.int32, L)
        for k in range(N_FEAT):
            row = tab_v[pl.ds(k * D, D)] + bvec
            plsc.store_scatter(tabt_v, [lane * N_FEAT + k], row)
        wsplats = [w_v[pl.ds(d * D, D)] for d in range(D)]

        n_s = jnp.where(wid < S_BIG, 7, 6)
        s0 = jnp.where(wid < S_BIG, wid * 7, S_BIG * 7 + (wid - S_BIG) * 6)

        def plane_body(s_i, carry):
            s = s0 + s_i

            def chunk_body(cb, ccarry):
                b0 = cb * CHUNK_B
                pltpu.sync_copy(ids_hbm.at[pl.ds(s * B_SZ + b0, CHUNK_B)], ids_v)
                pltpu.sync_copy(vals_hbm.at[pl.ds(s * B_SZ + b0, CHUNK_B)], vals_v)

                def group_body(vg, gcarry):
                    ids_vec = ids_v[pl.ds(vg * L, L)]
                    vals_vec = vals_v[pl.ds(vg * L, L)]
                    base = ((vg >> 6) << 13) + (((vg >> 3) & 7) << 7) + ((vg & 7) << 4)
                    for d in range(D):
                        col = plsc.load_gather(tabt_v, [ids_vec + d * N_FEAT])
                        col = col + vals_vec * wsplats[d]
                        off = base + (d // 8) * HALF + (d % 8) * 1024
                        out_v[pl.ds(off, L)] = col
                    return gcarry

                lax.fori_loop(0, CHUNK_B // L, group_body, 0)

                for dt in range(2):
                    dst = (s * 2 + dt) * (PLANE // 2) + cb * HALF
                    pltpu.sync_copy(out_v.at[pl.ds(dt * HALF, HALF)],
                                    out_hbm.at[pl.ds(dst, HALF)])
                return ccarry

            lax.fori_loop(0, CB, chunk_body, 0)
            return carry

        lax.fori_loop(0, n_s, plane_body, 0)

    return run


def kernel(feature_ids, feature_values, type_embed_table, value_proj_W, value_proj_b):
    batch, seq = feature_ids.shape
    ids = jnp.transpose(feature_ids, (1, 0)).reshape(seq * batch).astype(jnp.int32)
    vals = jnp.transpose(feature_values, (1, 2, 0)).reshape(seq * batch)
    tab = type_embed_table.reshape(N_FEAT * D).astype(jnp.float32)
    w = value_proj_W.reshape(D).astype(jnp.float32)
    wmat = jnp.broadcast_to(w[:, None], (D, D)).reshape(D * D)
    b = value_proj_b.reshape(D).astype(jnp.float32)
    out_flat = _build()(ids, vals, tab, wmat, b)
    out6 = out_flat.reshape(S_SZ, 2, 16, 8, 8, 128)
    return out6.transpose(2, 4, 5, 0, 1, 3).reshape(batch, seq, D)


# speedup vs baseline: 12.9164x; 1.1236x over previous
import functools

import jax
import jax.numpy as jnp
from jax import lax
from jax.experimental import pallas as pl
from jax.experimental.pallas import tpu as pltpu
from jax.experimental.pallas import tpu_sc as plsc

NC = 2
NS = 16
NW = NC * NS
L = 16

N_FEAT = 6
D = 16
B_SZ = 16384
S_SZ = 200

BW = 4
CB = B_SZ // (BW * 1024)
CHUNK_B = BW * 1024
PLANE = D * B_SZ
HALF = CHUNK_B * 8

S_BIG = S_SZ - 6 * NW


@functools.lru_cache(maxsize=None)
def _build():
    mesh = plsc.VectorSubcoreMesh(
        core_axis_name="c", subcore_axis_name="s", num_cores=NC, num_subcores=NS
    )

    @functools.partial(
        pl.kernel,
        out_type=jax.ShapeDtypeStruct((S_SZ * PLANE,), jnp.float32),
        mesh=mesh,
        scratch_types=[
            pltpu.VMEM((N_FEAT * D,), jnp.float32),
            pltpu.VMEM((D * N_FEAT,), jnp.float32),
            pltpu.VMEM((D * D,), jnp.float32),
            pltpu.VMEM((D,), jnp.float32),
            pltpu.VMEM((CHUNK_B,), jnp.int32),
            pltpu.VMEM((CHUNK_B,), jnp.float32),
            pltpu.VMEM((D * CHUNK_B,), jnp.float32),
        ],
        compiler_params=pltpu.CompilerParams(needs_layout_passes=False),
    )
    def run(ids_hbm, vals_hbm, tab_hbm, w_hbm, b_hbm, out_hbm,
            tab_v, tabt_v, w_v, b_v, ids_v, vals_v, out_v):
        wid = lax.axis_index("s") * NC + lax.axis_index("c")

        pltpu.sync_copy(tab_hbm, tab_v)
        pltpu.sync_copy(w_hbm, w_v)
        pltpu.sync_copy(b_hbm, b_v)

        bvec = b_v[...]
        lane = lax.iota(jnp.int32, L)
        for k in range(N_FEAT):
            row = tab_v[pl.ds(k * D, D)] + bvec
            plsc.store_scatter(tabt_v, [lane * N_FEAT + k], row)
        wsplats = [w_v[pl.ds(d * D, D)] for d in range(D)]

        n_s = jnp.where(wid < S_BIG, 7, 6)
        s0 = jnp.where(wid < S_BIG, wid * 7, S_BIG * 7 + (wid - S_BIG) * 6)

        def plane_body(s_i, carry):
            s = s0 + s_i

            def chunk_body(cb, ccarry):
                b0 = cb * CHUNK_B
                pltpu.sync_copy(ids_hbm.at[pl.ds(s * B_SZ + b0, CHUNK_B)], ids_v)
                pltpu.sync_copy(vals_hbm.at[pl.ds(s * B_SZ + b0, CHUNK_B)], vals_v)

                def group_body(vg, gcarry):
                    ids_vec = ids_v[pl.ds(vg * L, L)]
                    vals_vec = vals_v[pl.ds(vg * L, L)]
                    base = vg * L
                    for d in range(D):
                        col = plsc.load_gather(tabt_v, [ids_vec + d * N_FEAT])
                        col = col + vals_vec * wsplats[d]
                        out_v[pl.ds(base + d * CHUNK_B, L)] = col
                    return gcarry

                lax.fori_loop(0, CHUNK_B // L, group_body, 0)

                for d in range(D):
                    dst = (s * D + d) * B_SZ + b0
                    pltpu.sync_copy(out_v.at[pl.ds(d * CHUNK_B, CHUNK_B)],
                                    out_hbm.at[pl.ds(dst, CHUNK_B)])
                return ccarry

            lax.fori_loop(0, CB, chunk_body, 0)
            return carry

        lax.fori_loop(0, n_s, plane_body, 0)

    return run


def kernel(feature_ids, feature_values, type_embed_table, value_proj_W, value_proj_b):
    batch, seq = feature_ids.shape
    ids = jnp.transpose(feature_ids, (1, 0)).reshape(seq * batch).astype(jnp.int32)
    vals = jnp.transpose(feature_values, (1, 2, 0)).reshape(seq * batch)
    tab = type_embed_table.reshape(N_FEAT * D).astype(jnp.float32)
    w = value_proj_W.reshape(D).astype(jnp.float32)
    wmat = jnp.broadcast_to(w[:, None], (D, D)).reshape(D * D)
    b = value_proj_b.reshape(D).astype(jnp.float32)
    out_flat = _build()(ids, vals, tab, wmat, b)
    return out_flat.reshape(seq, D, batch).transpose(2, 0, 1)


# speedup vs baseline: 14.4389x; 1.1179x over previous
import functools

import jax
import jax.numpy as jnp
from jax import lax
from jax.experimental import pallas as pl
from jax.experimental.pallas import tpu as pltpu
from jax.experimental.pallas import tpu_sc as plsc

NC = 2
NS = 16
NW = NC * NS
L = 16

N_FEAT = 6
D = 16
B_SZ = 16384
S_SZ = 200

CHUNK_B = 2048
CB = B_SZ // CHUNK_B
PLANE = D * B_SZ

S_BIG = S_SZ - 6 * NW


@functools.lru_cache(maxsize=None)
def _build():
    mesh = plsc.VectorSubcoreMesh(
        core_axis_name="c", subcore_axis_name="s", num_cores=NC, num_subcores=NS
    )

    @functools.partial(
        pl.kernel,
        out_type=jax.ShapeDtypeStruct((S_SZ * PLANE,), jnp.float32),
        mesh=mesh,
        scratch_types=[
            pltpu.VMEM((N_FEAT * D,), jnp.float32),
            pltpu.VMEM((D * N_FEAT,), jnp.float32),
            pltpu.VMEM((D * D,), jnp.float32),
            pltpu.VMEM((D,), jnp.float32),
            pltpu.VMEM((2, CHUNK_B), jnp.int32),
            pltpu.VMEM((2, CHUNK_B), jnp.float32),
            pltpu.VMEM((2, D * CHUNK_B), jnp.float32),
            pltpu.SemaphoreType.DMA,
            pltpu.SemaphoreType.DMA,
            pltpu.SemaphoreType.DMA,
            pltpu.SemaphoreType.DMA,
        ],
        compiler_params=pltpu.CompilerParams(needs_layout_passes=False),
    )
    def run(ids_hbm, vals_hbm, tab_hbm, w_hbm, b_hbm, out_hbm,
            tab_v, tabt_v, w_v, b_v, ids_v, vals_v, out_v,
            sin0, sin1, sout0, sout1):
        sins = (sin0, sin1)
        souts = (sout0, sout1)
        wid = lax.axis_index("s") * NC + lax.axis_index("c")

        pltpu.sync_copy(tab_hbm, tab_v)
        pltpu.sync_copy(w_hbm, w_v)
        pltpu.sync_copy(b_hbm, b_v)

        bvec = b_v[...]
        lane = lax.iota(jnp.int32, L)
        for k in range(N_FEAT):
            row = tab_v[pl.ds(k * D, D)] + bvec
            plsc.store_scatter(tabt_v, [lane * N_FEAT + k], row)
        wsplats = [w_v[pl.ds(d * D, D)] for d in range(D)]

        n_s = jnp.where(wid < S_BIG, 7, 6)
        s0 = jnp.where(wid < S_BIG, wid * 7, S_BIG * 7 + (wid - S_BIG) * 6)
        n_chunks = n_s * CB

        def start_in(c, p):
            src = (s0 + c // CB) * B_SZ + (c % CB) * CHUNK_B
            pltpu.async_copy(ids_hbm.at[pl.ds(src, CHUNK_B)], ids_v.at[p], sins[p])
            pltpu.async_copy(vals_hbm.at[pl.ds(src, CHUNK_B)], vals_v.at[p], sins[p])

        def wait_in(p):
            pltpu.make_async_copy(ids_hbm.at[pl.ds(0, CHUNK_B)],
                                  ids_v.at[p], sins[p]).wait()
            pltpu.make_async_copy(vals_hbm.at[pl.ds(0, CHUNK_B)],
                                  vals_v.at[p], sins[p]).wait()

        def wait_out(p):
            pltpu.make_async_copy(out_hbm.at[pl.ds(0, D * CHUNK_B)],
                                  out_v.at[p], souts[p]).wait()

        start_in(0, 0)
        start_in(1, 1)

        def pair_body(c2, carry):
            for p in range(2):
                c = c2 * 2 + p
                s = s0 + c // CB
                b0 = (c % CB) * CHUNK_B
                wait_in(p)

                @pl.when(c >= 2)
                def _():
                    wait_out(p)

                def group_body(vg, gcarry):
                    ids_vec = ids_v[p, pl.ds(vg * L, L)]
                    vals_vec = vals_v[p, pl.ds(vg * L, L)]
                    base = vg * L
                    for d in range(D):
                        col = plsc.load_gather(tabt_v, [ids_vec + d * N_FEAT])
                        col = col + vals_vec * wsplats[d]
                        out_v[p, pl.ds(base + d * CHUNK_B, L)] = col
                    return gcarry

                lax.fori_loop(0, CHUNK_B // L, group_body, 0)

                for d in range(D):
                    pltpu.async_copy(
                        out_v.at[p, pl.ds(d * CHUNK_B, CHUNK_B)],
                        out_hbm.at[pl.ds((s * D + d) * B_SZ + b0, CHUNK_B)],
                        souts[p])

                @pl.when(c + 2 < n_chunks)
                def _():
                    start_in(c + 2, p)
            return carry

        lax.fori_loop(0, n_chunks // 2, pair_body, 0)
        wait_out(0)
        wait_out(1)

    return run


def kernel(feature_ids, feature_values, type_embed_table, value_proj_W, value_proj_b):
    batch, seq = feature_ids.shape
    ids = jnp.transpose(feature_ids, (1, 0)).reshape(seq * batch).astype(jnp.int32)
    vals = jnp.transpose(feature_values, (1, 2, 0)).reshape(seq * batch)
    tab = type_embed_table.reshape(N_FEAT * D).astype(jnp.float32)
    w = value_proj_W.reshape(D).astype(jnp.float32)
    wmat = jnp.broadcast_to(w[:, None], (D, D)).reshape(D * D)
    b = value_proj_b.reshape(D).astype(jnp.float32)
    out_flat = _build()(ids, vals, tab, wmat, b)
    return out_flat.reshape(seq, D, batch).transpose(2, 0, 1)


# speedup vs baseline: 14.4900x; 1.0035x over previous
import functools

import jax
import jax.numpy as jnp
from jax import lax
from jax.experimental import pallas as pl
from jax.experimental.pallas import tpu as pltpu
from jax.experimental.pallas import tpu_sc as plsc

NC = 2
NS = 16
NW = NC * NS
L = 16

N_FEAT = 6
D = 16
B_SZ = 16384
S_SZ = 200

CHUNK_B = 2048
CB = B_SZ // CHUNK_B
PLANE = D * B_SZ

S_BIG = S_SZ - 6 * NW


@functools.lru_cache(maxsize=None)
def _build():
    mesh = plsc.VectorSubcoreMesh(
        core_axis_name="c", subcore_axis_name="s", num_cores=NC, num_subcores=NS
    )

    @functools.partial(
        pl.kernel,
        out_type=jax.ShapeDtypeStruct((S_SZ * PLANE,), jnp.float32),
        mesh=mesh,
        scratch_types=[
            pltpu.VMEM((N_FEAT * D,), jnp.float32),
            pltpu.VMEM((D * N_FEAT,), jnp.float32),
            pltpu.VMEM((D * D,), jnp.float32),
            pltpu.VMEM((D,), jnp.float32),
            pltpu.VMEM((2, CHUNK_B), jnp.int32),
            pltpu.VMEM((2, CHUNK_B), jnp.float32),
            pltpu.VMEM((2, D * CHUNK_B), jnp.float32),
            pltpu.SemaphoreType.DMA,
            pltpu.SemaphoreType.DMA,
            pltpu.SemaphoreType.DMA,
            pltpu.SemaphoreType.DMA,
        ],
        compiler_params=pltpu.CompilerParams(needs_layout_passes=False),
    )
    def run(ids_hbm, vals_hbm, tab_hbm, w_hbm, b_hbm, out_hbm,
            tab_v, tabt_v, w_v, b_v, ids_v, vals_v, out_v,
            sin0, sin1, sout0, sout1):
        sins = (sin0, sin1)
        souts = (sout0, sout1)
        wid = lax.axis_index("s") * NC + lax.axis_index("c")

        pltpu.sync_copy(tab_hbm, tab_v)
        pltpu.sync_copy(w_hbm, w_v)
        pltpu.sync_copy(b_hbm, b_v)

        bvec = b_v[...]
        lane = lax.iota(jnp.int32, L)
        for k in range(N_FEAT):
            row = tab_v[pl.ds(k * D, D)] + bvec
            plsc.store_scatter(tabt_v, [lane * N_FEAT + k], row)
        wsplats = [w_v[pl.ds(d * D, D)] for d in range(D)]

        n_s = jnp.where(wid < S_BIG, 7, 6)
        s0 = jnp.where(wid < S_BIG, wid * 7, S_BIG * 7 + (wid - S_BIG) * 6)
        n_chunks = n_s * CB

        def start_in(c, p):
            src = (s0 + c // CB) * B_SZ + (c % CB) * CHUNK_B
            pltpu.async_copy(ids_hbm.at[pl.ds(src, CHUNK_B)], ids_v.at[p], sins[p])
            pltpu.async_copy(vals_hbm.at[pl.ds(src, CHUNK_B)], vals_v.at[p], sins[p])

        def wait_in(p):
            pltpu.make_async_copy(ids_hbm.at[pl.ds(0, CHUNK_B)],
                                  ids_v.at[p], sins[p]).wait()
            pltpu.make_async_copy(vals_hbm.at[pl.ds(0, CHUNK_B)],
                                  vals_v.at[p], sins[p]).wait()

        def wait_out(p):
            pltpu.make_async_copy(out_hbm.at[pl.ds(0, D * CHUNK_B)],
                                  out_v.at[p], souts[p]).wait()

        start_in(0, 0)
        start_in(1, 1)

        def pair_body(c2, carry):
            for p in range(2):
                c = c2 * 2 + p
                s = s0 + c // CB
                b0 = (c % CB) * CHUNK_B
                wait_in(p)

                @pl.when(c >= 2)
                def _():
                    wait_out(p)

                def group_body(vg, gcarry):
                    ids_vec = ids_v[p, pl.ds(vg * L, L)]
                    vals_vec = vals_v[p, pl.ds(vg * L, L)]
                    base = vg * L
                    for d in range(D):
                        col = plsc.load_gather(tabt_v, [ids_vec + d * N_FEAT])
                        col = col + vals_vec * wsplats[d]
                        out_v[p, pl.ds(base + d * CHUNK_B, L)] = col
                    return gcarry

                lax.fori_loop(0, CHUNK_B // L, group_body, 0)

                for d in range(D):
                    pltpu.async_copy(
                        out_v.at[p, pl.ds(d * CHUNK_B, CHUNK_B)],
                        out_hbm.at[pl.ds((s * D + d) * B_SZ + b0, CHUNK_B)],
                        souts[p])

                @pl.when(c + 2 < n_chunks)
                def _():
                    start_in(c + 2, p)
            return carry

        lax.fori_loop(0, n_chunks // 2, pair_body, 0)
        wait_out(0)
        wait_out(1)

    return run


def kernel(feature_ids, feature_values, type_embed_table, value_proj_W, value_proj_b):
    batch, seq = feature_ids.shape
    ids = jnp.transpose(feature_ids, (1, 0)).reshape(seq * batch).astype(jnp.int32)
    vals = jnp.transpose(feature_values, (1, 2, 0)).reshape(seq * batch)
    tab = type_embed_table.reshape(N_FEAT * D).astype(jnp.float32)
    w = value_proj_W.reshape(D).astype(jnp.float32)
    wmat = jnp.broadcast_to(w[:, None], (D, D)).reshape(D * D)
    b = value_proj_b.reshape(D).astype(jnp.float32)
    out_flat = _build()(ids, vals, tab, wmat, b)
    out6 = out_flat.reshape(seq, 2, 8, 16, 8, 128)
    return out6.transpose(3, 4, 5, 0, 1, 2).reshape(batch, seq, D)


# speedup vs baseline: 28.6109x; 1.9745x over previous
import functools

import jax
import jax.numpy as jnp
from jax import lax
from jax.experimental import pallas as pl
from jax.experimental.pallas import tpu as pltpu
from jax.experimental.pallas import tpu_sc as plsc

NC = 2
NS = 16
NW = NC * NS
L = 16

N_FEAT = 6
D = 16
B_SZ = 16384
S_SZ = 200

CHUNK_B = 2048
CB = B_SZ // CHUNK_B
PLANE = D * B_SZ

S_BIG = S_SZ - 6 * NW


@functools.lru_cache(maxsize=None)
def _build():
    mesh = plsc.VectorSubcoreMesh(
        core_axis_name="c", subcore_axis_name="s", num_cores=NC, num_subcores=NS
    )

    @functools.partial(
        pl.kernel,
        out_type=jax.ShapeDtypeStruct((S_SZ * PLANE,), jnp.float32),
        mesh=mesh,
        scratch_types=[
            pltpu.VMEM((N_FEAT * D,), jnp.float32),
            pltpu.VMEM((D * N_FEAT,), jnp.float32),
            pltpu.VMEM((D * D,), jnp.float32),
            pltpu.VMEM((D,), jnp.float32),
            pltpu.VMEM((2, CHUNK_B), jnp.int32),
            pltpu.VMEM((2, CHUNK_B), jnp.float32),
            pltpu.VMEM((2, D * CHUNK_B), jnp.float32),
            pltpu.SemaphoreType.DMA,
            pltpu.SemaphoreType.DMA,
            pltpu.SemaphoreType.DMA,
            pltpu.SemaphoreType.DMA,
        ],
        compiler_params=pltpu.CompilerParams(needs_layout_passes=False),
    )
    def run(ids_hbm, vals_hbm, tab_hbm, w_hbm, b_hbm, out_hbm,
            tab_v, tabt_v, w_v, b_v, ids_v, vals_v, out_v,
            sin0, sin1, sout0, sout1):
        sins = (sin0, sin1)
        souts = (sout0, sout1)
        wid = lax.axis_index("s") * NC + lax.axis_index("c")

        pltpu.sync_copy(tab_hbm, tab_v)
        pltpu.sync_copy(w_hbm, w_v)
        pltpu.sync_copy(b_hbm, b_v)

        bvec = b_v[...]
        lane = lax.iota(jnp.int32, L)
        for k in range(N_FEAT):
            row = tab_v[pl.ds(k * D, D)] + bvec
            plsc.store_scatter(tabt_v, [lane * N_FEAT + k], row)
        wsplats = [w_v[pl.ds(d * D, D)] for d in range(D)]

        n_s = jnp.where(wid < S_BIG, 7, 6)
        s0 = jnp.where(wid < S_BIG, wid * 7, S_BIG * 7 + (wid - S_BIG) * 6)
        n_chunks = n_s * CB

        def start_in(c, p):
            src = (s0 + c // CB) * B_SZ + (c % CB) * CHUNK_B
            pltpu.async_copy(ids_hbm.at[pl.ds(src, CHUNK_B)], ids_v.at[p], sins[p])
            pltpu.async_copy(vals_hbm.at[pl.ds(src, CHUNK_B)], vals_v.at[p], sins[p])

        def wait_in(p):
            pltpu.make_async_copy(ids_hbm.at[pl.ds(0, CHUNK_B)],
                                  ids_v.at[p], sins[p]).wait()
            pltpu.make_async_copy(vals_hbm.at[pl.ds(0, CHUNK_B)],
                                  vals_v.at[p], sins[p]).wait()

        def wait_out(p):
            pltpu.make_async_copy(out_hbm.at[pl.ds(0, D * CHUNK_B)],
                                  out_v.at[p], souts[p]).wait()

        start_in(0, 0)
        start_in(1, 1)

        def pair_body(c2, carry):
            for p in range(2):
                c = c2 * 2 + p
                s = s0 + c // CB
                b0 = (c % CB) * CHUNK_B
                wait_in(p)

                @pl.when(c >= 2)
                def _():
                    wait_out(p)

                def group_body(vg):
                    ids_vec = ids_v[p, pl.ds(vg * L, L)]
                    vals_vec = vals_v[p, pl.ds(vg * L, L)]
                    base = vg * L
                    cols = [plsc.load_gather(tabt_v, [ids_vec + d * N_FEAT])
                            for d in range(D)]
                    for d in range(D):
                        out_v[p, pl.ds(base + d * CHUNK_B, L)] = (
                            cols[d] + vals_vec * wsplats[d])

                plsc.parallel_loop(0, CHUNK_B // L, 1, unroll=2)(group_body)

                for d in range(D):
                    pltpu.async_copy(
                        out_v.at[p, pl.ds(d * CHUNK_B, CHUNK_B)],
                        out_hbm.at[pl.ds((s * D + d) * B_SZ + b0, CHUNK_B)],
                        souts[p])

                @pl.when(c + 2 < n_chunks)
                def _():
                    start_in(c + 2, p)
            return carry

        lax.fori_loop(0, n_chunks // 2, pair_body, 0)
        wait_out(0)
        wait_out(1)

    return run


def kernel(feature_ids, feature_values, type_embed_table, value_proj_W, value_proj_b):
    batch, seq = feature_ids.shape
    ids = jnp.transpose(feature_ids, (1, 0)).reshape(seq * batch).astype(jnp.int32)
    vals = jnp.transpose(feature_values, (1, 2, 0)).reshape(seq * batch)
    tab = type_embed_table.reshape(N_FEAT * D).astype(jnp.float32)
    w = value_proj_W.reshape(D).astype(jnp.float32)
    wmat = jnp.broadcast_to(w[:, None], (D, D)).reshape(D * D)
    b = value_proj_b.reshape(D).astype(jnp.float32)
    out_flat = _build()(ids, vals, tab, wmat, b)
    out6 = out_flat.reshape(seq, 2, 8, 16, 8, 128)
    return out6.transpose(3, 4, 5, 0, 1, 2).reshape(batch, seq, D)


# speedup vs baseline: 65.2274x; 2.2798x over previous
import functools

import jax
import jax.numpy as jnp
from jax import lax
from jax.experimental import pallas as pl
from jax.experimental.pallas import tpu as pltpu
from jax.experimental.pallas import tpu_sc as plsc

NC = 2
NS = 16
NW = NC * NS
L = 16

N_FEAT = 6
D = 16
B_SZ = 16384
S_SZ = 200

CHUNK_B = 2048
CB = B_SZ // CHUNK_B
PLANE = D * B_SZ

S_BIG = S_SZ - 6 * NW


@functools.lru_cache(maxsize=None)
def _build():
    mesh = plsc.VectorSubcoreMesh(
        core_axis_name="c", subcore_axis_name="s", num_cores=NC, num_subcores=NS
    )

    @functools.partial(
        pl.kernel,
        out_type=jax.ShapeDtypeStruct((S_SZ * D, B_SZ), jnp.float32),
        mesh=mesh,
        scratch_types=[
            pltpu.VMEM((N_FEAT * D,), jnp.float32),
            pltpu.VMEM((D * N_FEAT,), jnp.float32),
            pltpu.VMEM((D * D,), jnp.float32),
            pltpu.VMEM((D,), jnp.float32),
            pltpu.VMEM((2, CHUNK_B), jnp.int32),
            pltpu.VMEM((2, CHUNK_B), jnp.float32),
            pltpu.VMEM((2, D, CHUNK_B), jnp.float32),
            pltpu.SemaphoreType.DMA,
            pltpu.SemaphoreType.DMA,
            pltpu.SemaphoreType.DMA,
            pltpu.SemaphoreType.DMA,
        ],
        compiler_params=pltpu.CompilerParams(needs_layout_passes=False),
    )
    def run(ids_hbm, vals_hbm, tab_hbm, w_hbm, b_hbm, out_hbm,
            tab_v, tabt_v, w_v, b_v, ids_v, vals_v, out_v,
            sin0, sin1, sout0, sout1):
        sins = (sin0, sin1)
        souts = (sout0, sout1)
        wid = lax.axis_index("s") * NC + lax.axis_index("c")

        pltpu.sync_copy(tab_hbm, tab_v)
        pltpu.sync_copy(w_hbm, w_v)
        pltpu.sync_copy(b_hbm, b_v)

        bvec = b_v[...]
        lane = lax.iota(jnp.int32, L)
        for k in range(N_FEAT):
            row = tab_v[pl.ds(k * D, D)] + bvec
            plsc.store_scatter(tabt_v, [lane * N_FEAT + k], row)
        wsplats = [w_v[pl.ds(d * D, D)] for d in range(D)]

        n_s = jnp.where(wid < S_BIG, 7, 6)
        s0 = jnp.where(wid < S_BIG, wid * 7, S_BIG * 7 + (wid - S_BIG) * 6)
        n_chunks = n_s * CB

        def start_in(c, p):
            src = (s0 + c // CB) * B_SZ + (c % CB) * CHUNK_B
            pltpu.async_copy(ids_hbm.at[pl.ds(src, CHUNK_B)], ids_v.at[p], sins[p])
            pltpu.async_copy(vals_hbm.at[pl.ds(src, CHUNK_B)], vals_v.at[p], sins[p])

        def wait_in(p):
            pltpu.make_async_copy(ids_hbm.at[pl.ds(0, CHUNK_B)],
                                  ids_v.at[p], sins[p]).wait()
            pltpu.make_async_copy(vals_hbm.at[pl.ds(0, CHUNK_B)],
                                  vals_v.at[p], sins[p]).wait()

        def wait_out(p):
            pltpu.make_async_copy(out_hbm.at[pl.ds(0, D), pl.ds(0, CHUNK_B)],
                                  out_v.at[p], souts[p]).wait()

        start_in(0, 0)
        start_in(1, 1)

        def pair_body(c2, carry):
            for p in range(2):
                c = c2 * 2 + p
                s = s0 + c // CB
                b0 = (c % CB) * CHUNK_B
                wait_in(p)

                @pl.when(c >= 2)
                def _():
                    wait_out(p)

                def group_body(vg):
                    ids_vec = ids_v[p, pl.ds(vg * L, L)]
                    vals_vec = vals_v[p, pl.ds(vg * L, L)]
                    base = vg * L
                    cols = [plsc.load_gather(tabt_v, [ids_vec + d * N_FEAT])
                            for d in range(D)]
                    for d in range(D):
                        out_v[p, d, pl.ds(base, L)] = (
                            cols[d] + vals_vec * wsplats[d])

                plsc.parallel_loop(0, CHUNK_B // L, 1, unroll=2)(group_body)

                pltpu.async_copy(
                    out_v.at[p],
                    out_hbm.at[pl.ds(s * D, D), pl.ds(b0, CHUNK_B)],
                    souts[p])

                @pl.when(c + 2 < n_chunks)
                def _():
                    start_in(c + 2, p)
            return carry

        lax.fori_loop(0, n_chunks // 2, pair_body, 0)
        wait_out(0)
        wait_out(1)

    return run


def kernel(feature_ids, feature_values, type_embed_table, value_proj_W, value_proj_b):
    batch, seq = feature_ids.shape
    ids = jnp.transpose(feature_ids, (1, 0)).reshape(seq * batch).astype(jnp.int32)
    vals = jnp.transpose(feature_values, (1, 2, 0)).reshape(seq * batch)
    tab = type_embed_table.reshape(N_FEAT * D).astype(jnp.float32)
    w = value_proj_W.reshape(D).astype(jnp.float32)
    wmat = jnp.broadcast_to(w[:, None], (D, D)).reshape(D * D)
    b = value_proj_b.reshape(D).astype(jnp.float32)
    out2d = _build()(ids, vals, tab, wmat, b)
    out6 = out2d.reshape(seq, 2, 8, 16, 8, 128)
    return out6.transpose(3, 4, 5, 0, 1, 2).reshape(batch, seq, D)


# speedup vs baseline: 88.2767x; 1.3534x over previous
import functools

import jax
import jax.numpy as jnp
from jax import lax
from jax.experimental import pallas as pl
from jax.experimental.pallas import tpu as pltpu
from jax.experimental.pallas import tpu_sc as plsc

NC = 2
NS = 16
NW = NC * NS
L = 16

N_FEAT = 6
D = 16
B_SZ = 16384
S_SZ = 200

CHUNK_B = 2048
CB = B_SZ // CHUNK_B
PLANE = D * B_SZ

S_BIG = S_SZ - 6 * NW


@functools.lru_cache(maxsize=None)
def _build():
    mesh = plsc.VectorSubcoreMesh(
        core_axis_name="c", subcore_axis_name="s", num_cores=NC, num_subcores=NS
    )

    @functools.partial(
        pl.kernel,
        out_type=jax.ShapeDtypeStruct((S_SZ * D, B_SZ), jnp.float32),
        mesh=mesh,
        scratch_types=[
            pltpu.VMEM((N_FEAT * D,), jnp.float32),
            pltpu.VMEM((D * N_FEAT,), jnp.float32),
            pltpu.VMEM((D * D,), jnp.float32),
            pltpu.VMEM((D,), jnp.float32),
            pltpu.VMEM((2, CHUNK_B // 128, 128), jnp.int32),
            pltpu.VMEM((2, CHUNK_B), jnp.float32),
            pltpu.VMEM((2, D, CHUNK_B), jnp.float32),
            pltpu.SemaphoreType.DMA,
            pltpu.SemaphoreType.DMA,
            pltpu.SemaphoreType.DMA,
            pltpu.SemaphoreType.DMA,
        ],
        compiler_params=pltpu.CompilerParams(needs_layout_passes=False),
    )
    def run(ids_hbm, vals_hbm, tab_hbm, w_hbm, b_hbm, out_hbm,
            tab_v, tabt_v, w_v, b_v, ids_v, vals_v, out_v,
            sin0, sin1, sout0, sout1):
        sins = (sin0, sin1)
        souts = (sout0, sout1)
        wid = lax.axis_index("s") * NC + lax.axis_index("c")

        pltpu.sync_copy(tab_hbm, tab_v)
        pltpu.sync_copy(w_hbm, w_v)
        pltpu.sync_copy(b_hbm, b_v)

        bvec = b_v[...]
        lane = lax.iota(jnp.int32, L)
        for k in range(N_FEAT):
            row = tab_v[pl.ds(k * D, D)] + bvec
            plsc.store_scatter(tabt_v, [lane * N_FEAT + k], row)
        wsplats = [w_v[pl.ds(d * D, D)] for d in range(D)]

        n_s = jnp.where(wid < S_BIG, 7, 6)
        s0 = jnp.where(wid < S_BIG, wid * 7, S_BIG * 7 + (wid - S_BIG) * 6)
        n_chunks = n_s * CB

        def start_in(c, p):
            s = s0 + c // CB
            b0 = (c % CB) * CHUNK_B
            pltpu.async_copy(
                ids_hbm.at[s // 8, pl.ds(b0 // 128, CHUNK_B // 128), s % 8, :],
                ids_v.at[p], sins[p])
            pltpu.async_copy(vals_hbm.at[pl.ds(s * B_SZ + b0, CHUNK_B)],
                             vals_v.at[p], sins[p])

        def wait_in(p):
            pltpu.make_async_copy(
                ids_hbm.at[0, pl.ds(0, CHUNK_B // 128), 0, :],
                ids_v.at[p], sins[p]).wait()
            pltpu.make_async_copy(vals_hbm.at[pl.ds(0, CHUNK_B)],
                                  vals_v.at[p], sins[p]).wait()

        def wait_out(p):
            pltpu.make_async_copy(out_hbm.at[pl.ds(0, D), pl.ds(0, CHUNK_B)],
                                  out_v.at[p], souts[p]).wait()

        start_in(0, 0)
        start_in(1, 1)

        def pair_body(c2, carry):
            for p in range(2):
                c = c2 * 2 + p
                s = s0 + c // CB
                b0 = (c % CB) * CHUNK_B
                wait_in(p)

                @pl.when(c >= 2)
                def _():
                    wait_out(p)

                def group_body(vg):
                    ids_vec = ids_v[p, vg // 8, pl.ds((vg % 8) * L, L)]
                    vals_vec = vals_v[p, pl.ds(vg * L, L)]
                    base = vg * L
                    cols = [plsc.load_gather(tabt_v, [ids_vec + d * N_FEAT])
                            for d in range(D)]
                    for d in range(D):
                        out_v[p, d, pl.ds(base, L)] = (
                            cols[d] + vals_vec * wsplats[d])

                plsc.parallel_loop(0, CHUNK_B // L, 1, unroll=2)(group_body)

                pltpu.async_copy(
                    out_v.at[p],
                    out_hbm.at[pl.ds(s * D, D), pl.ds(b0, CHUNK_B)],
                    souts[p])

                @pl.when(c + 2 < n_chunks)
                def _():
                    start_in(c + 2, p)
            return carry

        lax.fori_loop(0, n_chunks // 2, pair_body, 0)
        wait_out(0)
        wait_out(1)

    return run


def kernel(feature_ids, feature_values, type_embed_table, value_proj_W, value_proj_b):
    batch, seq = feature_ids.shape
    ids4 = (jnp.transpose(feature_ids.astype(jnp.int32), (1, 0))
            .reshape(seq // 8, 8, batch // 128, 128).transpose(0, 2, 1, 3))
    vals = jnp.transpose(feature_values, (1, 2, 0)).reshape(seq * batch)
    tab = type_embed_table.reshape(N_FEAT * D).astype(jnp.float32)
    w = value_proj_W.reshape(D).astype(jnp.float32)
    wmat = jnp.broadcast_to(w[:, None], (D, D)).reshape(D * D)
    b = value_proj_b.reshape(D).astype(jnp.float32)
    out2d = _build()(ids4, vals, tab, wmat, b)
    out6 = out2d.reshape(seq, 2, 8, 16, 8, 128)
    return out6.transpose(3, 4, 5, 0, 1, 2).reshape(batch, seq, D)


# speedup vs baseline: 98.1152x; 1.1115x over previous
import functools

import jax
import jax.numpy as jnp
from jax import lax
from jax.experimental import pallas as pl
from jax.experimental.pallas import tpu as pltpu
from jax.experimental.pallas import tpu_sc as plsc

NC = 2
NS = 16
NW = NC * NS
L = 16

N_FEAT = 6
D = 16
B_SZ = 16384
S_SZ = 200

CHUNK_B = 2048
CB = B_SZ // CHUNK_B
PLANE = D * B_SZ

@functools.lru_cache(maxsize=None)
def _build():
    mesh = plsc.VectorSubcoreMesh(
        core_axis_name="c", subcore_axis_name="s", num_cores=NC, num_subcores=NS
    )

    @functools.partial(
        pl.kernel,
        out_type=jax.ShapeDtypeStruct((S_SZ * D, B_SZ), jnp.float32),
        mesh=mesh,
        scratch_types=[
            pltpu.VMEM((N_FEAT * D,), jnp.float32),
            pltpu.VMEM((D * N_FEAT,), jnp.float32),
            pltpu.VMEM((D * D,), jnp.float32),
            pltpu.VMEM((D,), jnp.float32),
            pltpu.VMEM((2, CHUNK_B // 128, 128), jnp.int32),
            pltpu.VMEM((2, CHUNK_B), jnp.float32),
            pltpu.VMEM((2, D, CHUNK_B), jnp.float32),
            pltpu.SemaphoreType.DMA,
            pltpu.SemaphoreType.DMA,
            pltpu.SemaphoreType.DMA,
            pltpu.SemaphoreType.DMA,
        ],
        compiler_params=pltpu.CompilerParams(needs_layout_passes=False),
    )
    def run(ids_hbm, vals_hbm, tab_hbm, w_hbm, b_hbm, out_hbm,
            tab_v, tabt_v, w_v, b_v, ids_v, vals_v, out_v,
            sin0, sin1, sout0, sout1):
        sins = (sin0, sin1)
        souts = (sout0, sout1)
        wid = lax.axis_index("s") * NC + lax.axis_index("c")

        pltpu.sync_copy(tab_hbm, tab_v)
        pltpu.sync_copy(w_hbm, w_v)
        pltpu.sync_copy(b_hbm, b_v)

        bvec = b_v[...]
        lane = lax.iota(jnp.int32, L)
        for k in range(N_FEAT):
            row = tab_v[pl.ds(k * D, D)] + bvec
            plsc.store_scatter(tabt_v, [lane * N_FEAT + k], row)
        wsplats = [w_v[pl.ds(d * D, D)] for d in range(D)]

        n_chunks = (S_SZ * CB) // NW
        c0 = wid * n_chunks

        def start_in(c, p):
            s = (c0 + c) // CB
            b0 = ((c0 + c) % CB) * CHUNK_B
            pltpu.async_copy(
                ids_hbm.at[s // 8, pl.ds(b0 // 128, CHUNK_B // 128), s % 8, :],
                ids_v.at[p], sins[p])
            pltpu.async_copy(vals_hbm.at[pl.ds(s * B_SZ + b0, CHUNK_B)],
                             vals_v.at[p], sins[p])

        def wait_in(p):
            pltpu.make_async_copy(
                ids_hbm.at[0, pl.ds(0, CHUNK_B // 128), 0, :],
                ids_v.at[p], sins[p]).wait()
            pltpu.make_async_copy(vals_hbm.at[pl.ds(0, CHUNK_B)],
                                  vals_v.at[p], sins[p]).wait()

        def wait_out(p):
            pltpu.make_async_copy(out_hbm.at[pl.ds(0, D), pl.ds(0, CHUNK_B)],
                                  out_v.at[p], souts[p]).wait()

        start_in(0, 0)
        start_in(1, 1)

        def pair_body(c2, carry):
            for p in range(2):
                c = c2 * 2 + p
                s = (c0 + c) // CB
                b0 = ((c0 + c) % CB) * CHUNK_B
                wait_in(p)

                @pl.when(c >= 2)
                def _():
                    wait_out(p)

                def group_body(vg):
                    ids_vec = ids_v[p, vg // 8, pl.ds((vg % 8) * L, L)]
                    vals_vec = vals_v[p, pl.ds(vg * L, L)]
                    base = vg * L
                    cols = [plsc.load_gather(tabt_v, [ids_vec + d * N_FEAT])
                            for d in range(D)]
                    for d in range(D):
                        out_v[p, d, pl.ds(base, L)] = (
                            cols[d] + vals_vec * wsplats[d])

                plsc.parallel_loop(0, CHUNK_B // L, 1, unroll=2)(group_body)

                pltpu.async_copy(
                    out_v.at[p],
                    out_hbm.at[pl.ds(s * D, D), pl.ds(b0, CHUNK_B)],
                    souts[p])

                @pl.when(c + 2 < n_chunks)
                def _():
                    start_in(c + 2, p)
            return carry

        lax.fori_loop(0, n_chunks // 2, pair_body, 0)
        wait_out(0)
        wait_out(1)

    return run


def kernel(feature_ids, feature_values, type_embed_table, value_proj_W, value_proj_b):
    batch, seq = feature_ids.shape
    ids4 = (jnp.transpose(feature_ids.astype(jnp.int32), (1, 0))
            .reshape(seq // 8, 8, batch // 128, 128).transpose(0, 2, 1, 3))
    vals = jnp.transpose(feature_values, (1, 2, 0)).reshape(seq * batch)
    tab = type_embed_table.reshape(N_FEAT * D).astype(jnp.float32)
    w = value_proj_W.reshape(D).astype(jnp.float32)
    wmat = jnp.broadcast_to(w[:, None], (D, D)).reshape(D * D)
    b = value_proj_b.reshape(D).astype(jnp.float32)
    out2d = _build()(ids4, vals, tab, wmat, b)
    out6 = out2d.reshape(seq, 2, 8, 16, 8, 128)
    return out6.transpose(3, 4, 5, 0, 1, 2).reshape(batch, seq, D)
